# baseline (speedup 1.0000x reference)
import jax
import jax.numpy as jnp
from jax import lax
from jax.experimental import pallas as pl
from jax.experimental.pallas import tpu as pltpu
from jax.experimental.pallas import tpu_sc as plsc

N_NODES = 10000
E_EDGES = 320000
D = 128
ROW_W = 144

NUM_CORES = 2
NUM_SUBCORES = 16
CHUNK = 50
EDGES_PER_TILE = E_EDGES // NUM_SUBCORES
NCHUNK = EDGES_PER_TILE // CHUNK
ROWS_PER_TILE = N_NODES // NUM_SUBCORES

NMSG = 4
NIDX = 8
UNROLL = 8
NITER = NCHUNK // UNROLL


def _sc_aggregate_body(x_hbm, sidx_hbm, didx_hbm, zrow_hbm, acc_hbm, *scr):
    sidxb = scr[0:NIDX]
    didxb = scr[NIDX:2 * NIDX]
    msg = scr[2 * NIDX:2 * NIDX + NMSG]
    o = 2 * NIDX + NMSG
    sem_is = scr[o:o + NIDX]
    sem_id = scr[o + NIDX:o + 2 * NIDX]
    sem_g = scr[o + 2 * NIDX:o + 2 * NIDX + NMSG]
    sem_s = scr[o + 2 * NIDX + NMSG:o + 2 * NIDX + 2 * NMSG]
    acc_sh = scr[o + 2 * NIDX + 2 * NMSG]

    c = lax.axis_index("c")
    s = lax.axis_index("s")
    rows = pl.ds(s * ROWS_PER_TILE, ROWS_PER_TILE)

    pltpu.sync_copy(zrow_hbm, acc_sh.at[rows, :])
    plsc.subcore_barrier()

    for q in range(5):
        pltpu.async_copy(sidx_hbm.at[c, s, q], sidxb[q], sem_is[q])
        pltpu.async_copy(didx_hbm.at[c, s, q], didxb[q], sem_id[q])
    for j in range(3):
        pltpu.make_async_copy(sidx_hbm.at[c, s, j], sidxb[j], sem_is[j]).wait()
        pltpu.async_copy(x_hbm.at[sidxb[j]], msg[j], sem_g[j])

    def step(i, carry):
        for k in range(UNROLL):
            ch = UNROLL * i + k
            m = k % NMSG
            q = k % NIDX
            m3 = (k + 3) % NMSG
            q3 = (k + 3) % NIDX
            q5 = (k + 5) % NIDX
            q7 = (k + 7) % NIDX

            pltpu.make_async_copy(x_hbm.at[sidxb[q]], msg[m], sem_g[m]).wait()
            pltpu.make_async_copy(didx_hbm.at[c, s, ch], didxb[q],
                                  sem_id[q]).wait()
            pltpu.async_copy(msg[m], acc_sh.at[didxb[q]], sem_s[m], add=True)

            def issue_gather():
                def wait_scatter():
                    pltpu.make_async_copy(msg[m3], acc_sh.at[didxb[q7]],
                                          sem_s[m3]).wait()
                if k == 0:
                    pl.when(i > 0)(wait_scatter)
                else:
                    wait_scatter()
                pltpu.make_async_copy(sidx_hbm.at[c, s, ch + 3], sidxb[q3],
                                      sem_is[q3]).wait()
                pltpu.async_copy(x_hbm.at[sidxb[q3]], msg[m3], sem_g[m3])

            if k <= 4:
                issue_gather()
            else:
                pl.when(i < NITER - 1)(issue_gather)

            def issue_idx():
                pltpu.async_copy(sidx_hbm.at[c, s, ch + 5], sidxb[q5],
                                 sem_is[q5])
                pltpu.async_copy(didx_hbm.at[c, s, ch + 5], didxb[q5],
                                 sem_id[q5])

            if k <= 2:
                issue_idx()
            else:
                pl.when(i < NITER - 1)(issue_idx)
        return carry

    lax.fori_loop(0, NITER, step, 0)
    for k in range(UNROLL - NMSG, UNROLL):
        m = k % NMSG
        q = k % NIDX
        pltpu.make_async_copy(msg[m], acc_sh.at[didxb[q]], sem_s[m]).wait()
    plsc.subcore_barrier()

    pltpu.sync_copy(acc_sh.at[rows, :], acc_hbm.at[c, rows, :])


def _tc_linear_body(accf_ref, wtf_ref, bf_ref,
                    accc_ref, wtc_ref, bc_ref,
                    outu_ref, outi_ref):
    cntf = accf_ref[0, :, D:D + 1]
    outu_ref[...] = (
        jnp.dot(accf_ref[0, :, :D] * (1.0 / jnp.maximum(cntf, 1.0)),
                wtf_ref[...], preferred_element_type=jnp.float32)
        + jnp.minimum(cntf, 1.0) * bf_ref[...])
    cntc = accc_ref[0, :, D:D + 1]
    outi_ref[...] = (
        jnp.dot(accc_ref[0, :, :D] * (1.0 / jnp.maximum(cntc, 1.0)),
                wtc_ref[...], preferred_element_type=jnp.float32)
        + jnp.minimum(cntc, 1.0) * bc_ref[...])


def kernel(x_user, x_item, edge_index_follows, edge_index_clicks,
           W_follows, b_follows, W_clicks, b_clicks):
    del x_item

    x_aug = jnp.zeros((N_NODES, ROW_W), jnp.float32)
    x_aug = x_aug.at[:, :D].set(x_user)
    x_aug = x_aug.at[:, D].set(1.0)
    sidx = jnp.stack([
        edge_index_follows[0].reshape(NUM_SUBCORES, NCHUNK, CHUNK),
        edge_index_clicks[0].reshape(NUM_SUBCORES, NCHUNK, CHUNK),
    ])
    didx = jnp.stack([
        edge_index_follows[1].reshape(NUM_SUBCORES, NCHUNK, CHUNK),
        edge_index_clicks[1].reshape(NUM_SUBCORES, NCHUNK, CHUNK),
    ])
    zrow = jnp.zeros((ROWS_PER_TILE, ROW_W), jnp.float32)

    mesh = plsc.VectorSubcoreMesh(core_axis_name="c", subcore_axis_name="s",
                                  num_cores=NUM_CORES,
                                  num_subcores=NUM_SUBCORES)
    acc = pl.kernel(
        _sc_aggregate_body,
        out_type=jax.ShapeDtypeStruct((NUM_CORES, N_NODES, ROW_W),
                                      jnp.float32),
        mesh=mesh,
        compiler_params=pltpu.CompilerParams(use_tc_tiling_on_sc=False),
        scratch_types=(
            [pltpu.VMEM((CHUNK,), jnp.int32)] * (2 * NIDX)
            + [pltpu.VMEM((CHUNK, ROW_W), jnp.float32)] * NMSG
            + [pltpu.SemaphoreType.DMA] * (2 * NIDX + 2 * NMSG)
            + [pltpu.VMEM_SHARED((N_NODES, ROW_W), jnp.float32)]
        ),
    )(x_aug, sidx, didx, zrow)

    ROW_BLK = 1000
    grid = (N_NODES // ROW_BLK,)
    accf_spec = pl.BlockSpec((1, ROW_BLK, ROW_W), lambda m: (0, m, 0))
    accc_spec = pl.BlockSpec((1, ROW_BLK, ROW_W), lambda m: (1, m, 0))
    w_spec = pl.BlockSpec((D, D), lambda m: (0, 0))
    b_spec = pl.BlockSpec((1, D), lambda m: (0, 0))
    out_spec = pl.BlockSpec((ROW_BLK, D), lambda m: (m, 0))
    out_user, out_item = pl.pallas_call(
        _tc_linear_body,
        grid=grid,
        in_specs=[accf_spec, w_spec, b_spec,
                  accc_spec, w_spec, b_spec],
        out_specs=[out_spec, out_spec],
        out_shape=[jax.ShapeDtypeStruct((N_NODES, D), jnp.float32),
                   jax.ShapeDtypeStruct((N_NODES, D), jnp.float32)],
    )(acc, W_follows.T, b_follows.reshape(1, D),
      acc, W_clicks.T, b_clicks.reshape(1, D))

    return (out_user, out_item)


# speedup vs baseline: 1.1194x; 1.1194x over previous
import jax
import jax.numpy as jnp
from jax import lax
from jax.experimental import pallas as pl
from jax.experimental.pallas import tpu as pltpu
from jax.experimental.pallas import tpu_sc as plsc

N_NODES = 10000
E_EDGES = 320000
D = 128

NUM_CORES = 2
NUM_SUBCORES = 16
CHUNK = 125
EDGES_PER_TILE = E_EDGES // NUM_SUBCORES
NCHUNK = EDGES_PER_TILE // CHUNK
ROWS_PER_TILE = N_NODES // NUM_SUBCORES
CNT_W = 8

NMSG = 2
NIDX = 4
UNROLL = 4
NITER = NCHUNK // UNROLL


def _sc_aggregate_body(x_hbm, sidx_hbm, didx_hbm, zrow_hbm, zcnt_hbm,
                       ones_hbm, acc_hbm, cnt_hbm, *scr):
    sidxb = scr[0:NIDX]
    didxb = scr[NIDX:2 * NIDX]
    msg = scr[2 * NIDX:2 * NIDX + NMSG]
    ones_v = scr[2 * NIDX + NMSG]
    o = 2 * NIDX + NMSG + 1
    sem_is = scr[o:o + NIDX]
    sem_id = scr[o + NIDX:o + 2 * NIDX]
    sem_g = scr[o + 2 * NIDX:o + 2 * NIDX + NMSG]
    sem_s = scr[o + 2 * NIDX + NMSG:o + 2 * NIDX + 2 * NMSG]
    sem_c = scr[o + 2 * NIDX + 2 * NMSG:o + 2 * NIDX + 3 * NMSG]
    acc_sh, cnt_sh = scr[o + 2 * NIDX + 3 * NMSG:]

    c = lax.axis_index("c")
    s = lax.axis_index("s")
    rows = pl.ds(s * ROWS_PER_TILE, ROWS_PER_TILE)

    pltpu.sync_copy(zrow_hbm, acc_sh.at[rows, :])
    pltpu.sync_copy(zcnt_hbm, cnt_sh.at[rows, :])
    pltpu.sync_copy(ones_hbm, ones_v)
    plsc.subcore_barrier()

    for q in range(3):
        pltpu.async_copy(sidx_hbm.at[c, s, q], sidxb[q], sem_is[q])
        pltpu.async_copy(didx_hbm.at[c, s, q], didxb[q], sem_id[q])
    pltpu.make_async_copy(sidx_hbm.at[c, s, 0], sidxb[0], sem_is[0]).wait()
    pltpu.async_copy(x_hbm.at[sidxb[0]], msg[0], sem_g[0])

    def step(i, carry):
        for k in range(UNROLL):
            ch = UNROLL * i + k
            m = k % NMSG
            q = k % NIDX
            m1 = (k + 1) % NMSG
            q1 = (k + 1) % NIDX
            q3 = (k + 3) % NIDX

            pltpu.make_async_copy(x_hbm.at[sidxb[q]], msg[m], sem_g[m]).wait()
            pltpu.make_async_copy(didx_hbm.at[c, s, ch], didxb[q],
                                  sem_id[q]).wait()
            pltpu.async_copy(msg[m], acc_sh.at[didxb[q]], sem_s[m], add=True)
            pltpu.async_copy(ones_v, cnt_sh.at[didxb[q]], sem_c[m], add=True)

            def issue_gather():
                def wait_scatter():
                    pltpu.make_async_copy(msg[m1], acc_sh.at[didxb[q3]],
                                          sem_s[m1]).wait()
                    pltpu.make_async_copy(ones_v, cnt_sh.at[didxb[q3]],
                                          sem_c[m1]).wait()
                if k == 0:
                    pl.when(i > 0)(wait_scatter)
                else:
                    wait_scatter()
                pltpu.make_async_copy(sidx_hbm.at[c, s, ch + 1], sidxb[q1],
                                      sem_is[q1]).wait()
                pltpu.async_copy(x_hbm.at[sidxb[q1]], msg[m1], sem_g[m1])

            if k <= 2:
                issue_gather()
            else:
                pl.when(i < NITER - 1)(issue_gather)

            def issue_idx():
                pltpu.async_copy(sidx_hbm.at[c, s, ch + 3], sidxb[q3],
                                 sem_is[q3])
                pltpu.async_copy(didx_hbm.at[c, s, ch + 3], didxb[q3],
                                 sem_id[q3])

            if k == 0:
                issue_idx()
            else:
                pl.when(i < NITER - 1)(issue_idx)
        return carry

    lax.fori_loop(0, NITER, step, 0)
    for ch_tail in (NCHUNK - 2, NCHUNK - 1):
        m = ch_tail % NMSG
        q = ch_tail % NIDX
        pltpu.make_async_copy(msg[m], acc_sh.at[didxb[q]], sem_s[m]).wait()
        pltpu.make_async_copy(ones_v, cnt_sh.at[didxb[q]], sem_c[m]).wait()
    plsc.subcore_barrier()

    pltpu.sync_copy(acc_sh.at[rows, :], acc_hbm.at[c, rows, :])
    pltpu.sync_copy(cnt_sh.at[rows, :], cnt_hbm.at[c, rows, :])


def _tc_linear_body(accf_ref, cntf_ref, wtf_ref, bf_ref,
                    accc_ref, cntc_ref, wtc_ref, bc_ref,
                    outu_ref, outi_ref):
    cntf = cntf_ref[0, :, 0:1]
    outu_ref[...] = (
        jnp.dot(accf_ref[0] * (1.0 / jnp.maximum(cntf, 1.0)), wtf_ref[...],
                preferred_element_type=jnp.float32)
        + jnp.minimum(cntf, 1.0) * bf_ref[...])
    cntc = cntc_ref[0, :, 0:1]
    outi_ref[...] = (
        jnp.dot(accc_ref[0] * (1.0 / jnp.maximum(cntc, 1.0)), wtc_ref[...],
                preferred_element_type=jnp.float32)
        + jnp.minimum(cntc, 1.0) * bc_ref[...])


def kernel(x_user, x_item, edge_index_follows, edge_index_clicks,
           W_follows, b_follows, W_clicks, b_clicks):
    del x_item

    sidx = jnp.stack([
        edge_index_follows[0].reshape(NUM_SUBCORES, NCHUNK, CHUNK),
        edge_index_clicks[0].reshape(NUM_SUBCORES, NCHUNK, CHUNK),
    ])
    didx = jnp.stack([
        edge_index_follows[1].reshape(NUM_SUBCORES, NCHUNK, CHUNK),
        edge_index_clicks[1].reshape(NUM_SUBCORES, NCHUNK, CHUNK),
    ])
    zrow = jnp.zeros((ROWS_PER_TILE, D), jnp.float32)
    zcnt = jnp.zeros((ROWS_PER_TILE, CNT_W), jnp.float32)
    ones = jnp.ones((CHUNK, CNT_W), jnp.float32)

    mesh = plsc.VectorSubcoreMesh(core_axis_name="c", subcore_axis_name="s",
                                  num_cores=NUM_CORES,
                                  num_subcores=NUM_SUBCORES)
    acc, cnt = pl.kernel(
        _sc_aggregate_body,
        out_type=[
            jax.ShapeDtypeStruct((NUM_CORES, N_NODES, D), jnp.float32),
            jax.ShapeDtypeStruct((NUM_CORES, N_NODES, CNT_W), jnp.float32),
        ],
        mesh=mesh,
        compiler_params=pltpu.CompilerParams(use_tc_tiling_on_sc=False),
        scratch_types=(
            [pltpu.VMEM((CHUNK,), jnp.int32)] * (2 * NIDX)
            + [pltpu.VMEM((CHUNK, D), jnp.float32)] * NMSG
            + [pltpu.VMEM((CHUNK, CNT_W), jnp.float32)]
            + [pltpu.SemaphoreType.DMA] * (2 * NIDX + 3 * NMSG)
            + [pltpu.VMEM_SHARED((N_NODES, D), jnp.float32),
               pltpu.VMEM_SHARED((N_NODES, CNT_W), jnp.float32)]
        ),
    )(x_user, sidx, didx, zrow, zcnt, ones)

    ROW_BLK = 1000
    grid = (N_NODES // ROW_BLK,)
    accf_spec = pl.BlockSpec((1, ROW_BLK, D), lambda m: (0, m, 0))
    accc_spec = pl.BlockSpec((1, ROW_BLK, D), lambda m: (1, m, 0))
    cntf_spec = pl.BlockSpec((1, ROW_BLK, CNT_W), lambda m: (0, m, 0))
    cntc_spec = pl.BlockSpec((1, ROW_BLK, CNT_W), lambda m: (1, m, 0))
    w_spec = pl.BlockSpec((D, D), lambda m: (0, 0))
    b_spec = pl.BlockSpec((1, D), lambda m: (0, 0))
    out_spec = pl.BlockSpec((ROW_BLK, D), lambda m: (m, 0))
    out_user, out_item = pl.pallas_call(
        _tc_linear_body,
        grid=grid,
        in_specs=[accf_spec, cntf_spec, w_spec, b_spec,
                  accc_spec, cntc_spec, w_spec, b_spec],
        out_specs=[out_spec, out_spec],
        out_shape=[jax.ShapeDtypeStruct((N_NODES, D), jnp.float32),
                   jax.ShapeDtypeStruct((N_NODES, D), jnp.float32)],
    )(acc, cnt, W_follows.T, b_follows.reshape(1, D),
      acc, cnt, W_clicks.T, b_clicks.reshape(1, D))

    return (out_user, out_item)


# speedup vs baseline: 1.2081x; 1.0792x over previous
import jax
import jax.numpy as jnp
from jax import lax
from jax.experimental import pallas as pl
from jax.experimental.pallas import tpu as pltpu
from jax.experimental.pallas import tpu_sc as plsc

N_NODES = 10000
E_EDGES = 320000
D = 128

NUM_CORES = 2
NUM_SUBCORES = 16
CHUNK = 125
EDGES_PER_TILE = E_EDGES // NUM_SUBCORES
NCHUNK = EDGES_PER_TILE // CHUNK
ROWS_PER_TILE = N_NODES // NUM_SUBCORES
CNT_W = 8

NMSG = 2
NIDX = 4
UNROLL = 4
NITER = NCHUNK // UNROLL


def _sc_aggregate_body(x_hbm, eif_hbm, eic_hbm, zrow_hbm, zcnt_hbm,
                       ones_hbm, acc_hbm, cnt_hbm, *scr):
    sidxb = scr[0:NIDX]
    didxb = scr[NIDX:2 * NIDX]
    msg = scr[2 * NIDX:2 * NIDX + NMSG]
    ones_v = scr[2 * NIDX + NMSG]
    o = 2 * NIDX + NMSG + 1
    sem_is = scr[o:o + NIDX]
    sem_id = scr[o + NIDX:o + 2 * NIDX]
    sem_g = scr[o + 2 * NIDX:o + 2 * NIDX + NMSG]
    sem_s = scr[o + 2 * NIDX + NMSG:o + 2 * NIDX + 2 * NMSG]
    sem_c = scr[o + 2 * NIDX + 2 * NMSG:o + 2 * NIDX + 3 * NMSG]
    acc_sh, cnt_sh = scr[o + 2 * NIDX + 3 * NMSG:]

    c = lax.axis_index("c")
    s = lax.axis_index("s")
    rows = pl.ds(s * ROWS_PER_TILE, ROWS_PER_TILE)

    pltpu.sync_copy(zrow_hbm, acc_sh.at[rows, :])
    pltpu.sync_copy(zcnt_hbm, cnt_sh.at[rows, :])
    pltpu.sync_copy(ones_hbm, ones_v)
    plsc.subcore_barrier()


    def run_pipeline(e_hbm):
        def sidx_src(ch):
            return e_hbm.at[0, s, ch]

        def didx_src(ch):
            return e_hbm.at[1, s, ch]

        for q in range(3):
            pltpu.async_copy(sidx_src(q), sidxb[q], sem_is[q])
            pltpu.async_copy(didx_src(q), didxb[q], sem_id[q])
        pltpu.make_async_copy(sidx_src(0), sidxb[0], sem_is[0]).wait()
        pltpu.async_copy(x_hbm.at[sidxb[0]], msg[0], sem_g[0])

        def step(i, carry):
            for k in range(UNROLL):
                ch = UNROLL * i + k
                m = k % NMSG
                q = k % NIDX
                m1 = (k + 1) % NMSG
                q1 = (k + 1) % NIDX
                q3 = (k + 3) % NIDX

                pltpu.make_async_copy(x_hbm.at[sidxb[q]], msg[m],
                                      sem_g[m]).wait()
                pltpu.make_async_copy(didx_src(ch), didxb[q], sem_id[q]).wait()
                pltpu.async_copy(msg[m], acc_sh.at[didxb[q]], sem_s[m],
                                 add=True)
                pltpu.async_copy(ones_v, cnt_sh.at[didxb[q]], sem_c[m],
                                 add=True)

                def issue_gather():
                    def wait_scatter():
                        pltpu.make_async_copy(msg[m1], acc_sh.at[didxb[q3]],
                                              sem_s[m1]).wait()
                        pltpu.make_async_copy(ones_v, cnt_sh.at[didxb[q3]],
                                              sem_c[m1]).wait()
                    if k == 0:
                        pl.when(i > 0)(wait_scatter)
                    else:
                        wait_scatter()
                    pltpu.make_async_copy(sidx_src(ch + 1), sidxb[q1],
                                          sem_is[q1]).wait()
                    pltpu.async_copy(x_hbm.at[sidxb[q1]], msg[m1], sem_g[m1])

                if k <= 2:
                    issue_gather()
                else:
                    pl.when(i < NITER - 1)(issue_gather)

                def issue_idx():
                    pltpu.async_copy(sidx_src(ch + 3), sidxb[q3], sem_is[q3])
                    pltpu.async_copy(didx_src(ch + 3), didxb[q3], sem_id[q3])

                if k == 0:
                    issue_idx()
                else:
                    pl.when(i < NITER - 1)(issue_idx)
            return carry

        lax.fori_loop(0, NITER, step, 0)
        for ch_tail in (NCHUNK - 2, NCHUNK - 1):
            m = ch_tail % NMSG
            q = ch_tail % NIDX
            pltpu.make_async_copy(msg[m], acc_sh.at[didxb[q]], sem_s[m]).wait()
            pltpu.make_async_copy(ones_v, cnt_sh.at[didxb[q]], sem_c[m]).wait()

    pl.when(c == 0)(lambda: run_pipeline(eif_hbm))
    pl.when(c == 1)(lambda: run_pipeline(eic_hbm))
    plsc.subcore_barrier()

    pltpu.sync_copy(acc_sh.at[rows, :], acc_hbm.at[c, rows, :])
    pltpu.sync_copy(cnt_sh.at[rows, :], cnt_hbm.at[c, rows, :])


def _tc_linear_body(accf_ref, cntf_ref, wtf_ref, bf_ref,
                    accc_ref, cntc_ref, wtc_ref, bc_ref,
                    outu_ref, outi_ref):
    cntf = cntf_ref[0, :, 0:1]
    outu_ref[...] = (
        jnp.dot(accf_ref[0] * (1.0 / jnp.maximum(cntf, 1.0)), wtf_ref[...],
                preferred_element_type=jnp.float32)
        + jnp.minimum(cntf, 1.0) * bf_ref[...])
    cntc = cntc_ref[0, :, 0:1]
    outi_ref[...] = (
        jnp.dot(accc_ref[0] * (1.0 / jnp.maximum(cntc, 1.0)), wtc_ref[...],
                preferred_element_type=jnp.float32)
        + jnp.minimum(cntc, 1.0) * bc_ref[...])


def kernel(x_user, x_item, edge_index_follows, edge_index_clicks,
           W_follows, b_follows, W_clicks, b_clicks):
    del x_item

    eif = edge_index_follows.reshape(2, NUM_SUBCORES, NCHUNK, CHUNK)
    eic = edge_index_clicks.reshape(2, NUM_SUBCORES, NCHUNK, CHUNK)
    zrow = jnp.zeros((ROWS_PER_TILE, D), jnp.float32)
    zcnt = jnp.zeros((ROWS_PER_TILE, CNT_W), jnp.float32)
    ones = jnp.ones((CHUNK, CNT_W), jnp.float32)

    mesh = plsc.VectorSubcoreMesh(core_axis_name="c", subcore_axis_name="s",
                                  num_cores=NUM_CORES,
                                  num_subcores=NUM_SUBCORES)
    acc, cnt = pl.kernel(
        _sc_aggregate_body,
        out_type=[
            jax.ShapeDtypeStruct((NUM_CORES, N_NODES, D), jnp.float32),
            jax.ShapeDtypeStruct((NUM_CORES, N_NODES, CNT_W), jnp.float32),
        ],
        mesh=mesh,
        compiler_params=pltpu.CompilerParams(use_tc_tiling_on_sc=False),
        scratch_types=(
            [pltpu.VMEM((CHUNK,), jnp.int32)] * (2 * NIDX)
            + [pltpu.VMEM((CHUNK, D), jnp.float32)] * NMSG
            + [pltpu.VMEM((CHUNK, CNT_W), jnp.float32)]
            + [pltpu.SemaphoreType.DMA] * (2 * NIDX + 3 * NMSG)
            + [pltpu.VMEM_SHARED((N_NODES, D), jnp.float32),
               pltpu.VMEM_SHARED((N_NODES, CNT_W), jnp.float32)]
        ),
    )(x_user, eif, eic, zrow, zcnt, ones)

    ROW_BLK = 1000
    grid = (N_NODES // ROW_BLK,)
    accf_spec = pl.BlockSpec((1, ROW_BLK, D), lambda m: (0, m, 0))
    accc_spec = pl.BlockSpec((1, ROW_BLK, D), lambda m: (1, m, 0))
    cntf_spec = pl.BlockSpec((1, ROW_BLK, CNT_W), lambda m: (0, m, 0))
    cntc_spec = pl.BlockSpec((1, ROW_BLK, CNT_W), lambda m: (1, m, 0))
    w_spec = pl.BlockSpec((D, D), lambda m: (0, 0))
    b_spec = pl.BlockSpec((1, D), lambda m: (0, 0))
    out_spec = pl.BlockSpec((ROW_BLK, D), lambda m: (m, 0))
    out_user, out_item = pl.pallas_call(
        _tc_linear_body,
        grid=grid,
        in_specs=[accf_spec, cntf_spec, w_spec, b_spec,
                  accc_spec, cntc_spec, w_spec, b_spec],
        out_specs=[out_spec, out_spec],
        out_shape=[jax.ShapeDtypeStruct((N_NODES, D), jnp.float32),
                   jax.ShapeDtypeStruct((N_NODES, D), jnp.float32)],
    )(acc, cnt, W_follows.T, b_follows.reshape(1, D),
      acc, cnt, W_clicks.T, b_clicks.reshape(1, D))

    return (out_user, out_item)


# speedup vs baseline: 1.2289x; 1.0173x over previous
import jax
import jax.numpy as jnp
from jax import lax
from jax.experimental import pallas as pl
from jax.experimental.pallas import tpu as pltpu
from jax.experimental.pallas import tpu_sc as plsc

N_NODES = 10000
E_EDGES = 320000
D = 128

NUM_CORES = 2
NUM_SUBCORES = 16
CHUNK = 125
EDGES_PER_TILE = E_EDGES // NUM_SUBCORES
NCHUNK = EDGES_PER_TILE // CHUNK
ROWS_PER_TILE = N_NODES // NUM_SUBCORES
CNT_W = 8

NMSG = 2
NIDX = 4
UNROLL = 4
NITER = NCHUNK // UNROLL


def _sc_aggregate_body(x_hbm, eif_hbm, eic_hbm, zrow_hbm, zcnt_hbm,
                       ones_hbm, acc_hbm, cnt_hbm, *scr):
    sidxb = scr[0:NIDX]
    didxb = scr[NIDX:2 * NIDX]
    msg = scr[2 * NIDX:2 * NIDX + NMSG]
    ones_v = scr[2 * NIDX + NMSG]
    o = 2 * NIDX + NMSG + 1
    sem_is = scr[o:o + NIDX]
    sem_id = scr[o + NIDX:o + 2 * NIDX]
    sem_g = scr[o + 2 * NIDX:o + 2 * NIDX + NMSG]
    sem_s = scr[o + 2 * NIDX + NMSG:o + 2 * NIDX + 2 * NMSG]
    sem_c = scr[o + 2 * NIDX + 2 * NMSG:o + 2 * NIDX + 3 * NMSG]
    acc_sh, cnt_sh = scr[o + 2 * NIDX + 3 * NMSG:]

    c = lax.axis_index("c")
    s = lax.axis_index("s")
    rows = pl.ds(s * ROWS_PER_TILE, ROWS_PER_TILE)

    pltpu.async_copy(zrow_hbm, acc_sh.at[rows, :], sem_g[1])
    pltpu.async_copy(zcnt_hbm, cnt_sh.at[rows, :], sem_s[1])
    pltpu.sync_copy(ones_hbm, ones_v)


    def run_pipeline(e_hbm):
        def sidx_src(ch):
            return e_hbm.at[0, s, ch]

        def didx_src(ch):
            return e_hbm.at[1, s, ch]

        for q in range(3):
            pltpu.async_copy(sidx_src(q), sidxb[q], sem_is[q])
            pltpu.async_copy(didx_src(q), didxb[q], sem_id[q])
        pltpu.make_async_copy(sidx_src(0), sidxb[0], sem_is[0]).wait()
        pltpu.async_copy(x_hbm.at[sidxb[0]], msg[0], sem_g[0])
        pltpu.make_async_copy(zrow_hbm, acc_sh.at[rows, :], sem_g[1]).wait()
        pltpu.make_async_copy(zcnt_hbm, cnt_sh.at[rows, :], sem_s[1]).wait()
        plsc.subcore_barrier()

        def step(i, carry):
            for k in range(UNROLL):
                ch = UNROLL * i + k
                m = k % NMSG
                q = k % NIDX
                m1 = (k + 1) % NMSG
                q1 = (k + 1) % NIDX
                q3 = (k + 3) % NIDX

                pltpu.make_async_copy(x_hbm.at[sidxb[q]], msg[m],
                                      sem_g[m]).wait()
                pltpu.make_async_copy(didx_src(ch), didxb[q], sem_id[q]).wait()
                pltpu.async_copy(msg[m], acc_sh.at[didxb[q]], sem_s[m],
                                 add=True)
                pltpu.async_copy(ones_v, cnt_sh.at[didxb[q]], sem_c[m],
                                 add=True)

                def issue_gather():
                    def wait_scatter():
                        pltpu.make_async_copy(msg[m1], acc_sh.at[didxb[q3]],
                                              sem_s[m1]).wait()
                        pltpu.make_async_copy(ones_v, cnt_sh.at[didxb[q3]],
                                              sem_c[m1]).wait()
                    if k == 0:
                        pl.when(i > 0)(wait_scatter)
                    else:
                        wait_scatter()
                    pltpu.make_async_copy(sidx_src(ch + 1), sidxb[q1],
                                          sem_is[q1]).wait()
                    pltpu.async_copy(x_hbm.at[sidxb[q1]], msg[m1], sem_g[m1])

                if k <= 2:
                    issue_gather()
                else:
                    pl.when(i < NITER - 1)(issue_gather)

                def issue_idx():
                    pltpu.async_copy(sidx_src(ch + 3), sidxb[q3], sem_is[q3])
                    pltpu.async_copy(didx_src(ch + 3), didxb[q3], sem_id[q3])

                if k == 0:
                    issue_idx()
                else:
                    pl.when(i < NITER - 1)(issue_idx)
            return carry

        lax.fori_loop(0, NITER, step, 0)
        for ch_tail in (NCHUNK - 2, NCHUNK - 1):
            m = ch_tail % NMSG
            q = ch_tail % NIDX
            pltpu.make_async_copy(msg[m], acc_sh.at[didxb[q]], sem_s[m]).wait()
            pltpu.make_async_copy(ones_v, cnt_sh.at[didxb[q]], sem_c[m]).wait()

    pl.when(c == 0)(lambda: run_pipeline(eif_hbm))
    pl.when(c == 1)(lambda: run_pipeline(eic_hbm))
    plsc.subcore_barrier()

    pltpu.sync_copy(acc_sh.at[rows, :], acc_hbm.at[c, rows, :])
    pltpu.sync_copy(cnt_sh.at[rows, :], cnt_hbm.at[c, rows, :])


def _tc_linear_body(accf_ref, cntf_ref, wtf_ref, bf_ref,
                    accc_ref, cntc_ref, wtc_ref, bc_ref,
                    outu_ref, outi_ref):
    cntf = cntf_ref[0, :, 0:1]
    outu_ref[...] = (
        jnp.dot(accf_ref[0] * (1.0 / jnp.maximum(cntf, 1.0)), wtf_ref[...],
                preferred_element_type=jnp.float32)
        + jnp.minimum(cntf, 1.0) * bf_ref[...])
    cntc = cntc_ref[0, :, 0:1]
    outi_ref[...] = (
        jnp.dot(accc_ref[0] * (1.0 / jnp.maximum(cntc, 1.0)), wtc_ref[...],
                preferred_element_type=jnp.float32)
        + jnp.minimum(cntc, 1.0) * bc_ref[...])


def kernel(x_user, x_item, edge_index_follows, edge_index_clicks,
           W_follows, b_follows, W_clicks, b_clicks):
    del x_item

    eif = edge_index_follows.reshape(2, NUM_SUBCORES, NCHUNK, CHUNK)
    eic = edge_index_clicks.reshape(2, NUM_SUBCORES, NCHUNK, CHUNK)
    zrow = jnp.zeros((ROWS_PER_TILE, D), jnp.float32)
    zcnt = jnp.zeros((ROWS_PER_TILE, CNT_W), jnp.float32)
    ones = jnp.ones((CHUNK, CNT_W), jnp.float32)

    mesh = plsc.VectorSubcoreMesh(core_axis_name="c", subcore_axis_name="s",
                                  num_cores=NUM_CORES,
                                  num_subcores=NUM_SUBCORES)
    acc, cnt = pl.kernel(
        _sc_aggregate_body,
        out_type=[
            jax.ShapeDtypeStruct((NUM_CORES, N_NODES, D), jnp.float32),
            jax.ShapeDtypeStruct((NUM_CORES, N_NODES, CNT_W), jnp.float32),
        ],
        mesh=mesh,
        compiler_params=pltpu.CompilerParams(use_tc_tiling_on_sc=False),
        scratch_types=(
            [pltpu.VMEM((CHUNK,), jnp.int32)] * (2 * NIDX)
            + [pltpu.VMEM((CHUNK, D), jnp.float32)] * NMSG
            + [pltpu.VMEM((CHUNK, CNT_W), jnp.float32)]
            + [pltpu.SemaphoreType.DMA] * (2 * NIDX + 3 * NMSG)
            + [pltpu.VMEM_SHARED((N_NODES, D), jnp.float32),
               pltpu.VMEM_SHARED((N_NODES, CNT_W), jnp.float32)]
        ),
    )(x_user, eif, eic, zrow, zcnt, ones)

    ROW_BLK = 2000
    grid = (N_NODES // ROW_BLK,)
    accf_spec = pl.BlockSpec((1, ROW_BLK, D), lambda m: (0, m, 0))
    accc_spec = pl.BlockSpec((1, ROW_BLK, D), lambda m: (1, m, 0))
    cntf_spec = pl.BlockSpec((1, ROW_BLK, CNT_W), lambda m: (0, m, 0))
    cntc_spec = pl.BlockSpec((1, ROW_BLK, CNT_W), lambda m: (1, m, 0))
    w_spec = pl.BlockSpec((D, D), lambda m: (0, 0))
    b_spec = pl.BlockSpec((1, D), lambda m: (0, 0))
    out_spec = pl.BlockSpec((ROW_BLK, D), lambda m: (m, 0))
    out_user, out_item = pl.pallas_call(
        _tc_linear_body,
        grid=grid,
        in_specs=[accf_spec, cntf_spec, w_spec, b_spec,
                  accc_spec, cntc_spec, w_spec, b_spec],
        out_specs=[out_spec, out_spec],
        out_shape=[jax.ShapeDtypeStruct((N_NODES, D), jnp.float32),
                   jax.ShapeDtypeStruct((N_NODES, D), jnp.float32)],
    )(acc, cnt, W_follows.T, b_follows.reshape(1, D),
      acc, cnt, W_clicks.T, b_clicks.reshape(1, D))

    return (out_user, out_item)
